# baseline (speedup 1.0000x reference)
import functools

import jax
import jax.numpy as jnp
import numpy as np
from jax import lax
from jax.experimental import pallas as pl
from jax.experimental.pallas import tpu as pltpu
from jax.experimental.pallas import tpu_sc as plsc

N = 10000
NP = 10016
VOCAB = 100
HID = 128
NH = 8
HD = 16
E_REAL = 330000
K = 128
NCHUNK = 1408
EPH = NCHUNK * K
CPW = NCHUNK // 16
NH2 = N // 2
NSP = NH2 + 8
RPT = 313
WB = 312

_NEG = -1e30


_mesh = plsc.VectorSubcoreMesh(core_axis_name="c", subcore_axis_name="s")


@functools.partial(
    pl.kernel,
    mesh=_mesh,
    compiler_params=pltpu.CompilerParams(use_tc_tiling_on_sc=False),
    out_type=[
        jax.ShapeDtypeStruct((2, NSP, HID + 16), jnp.float32),
    ],
    scratch_types=[
        pltpu.VMEM((CPW, K), jnp.int32),
        pltpu.VMEM((2, K), jnp.int32),
        pltpu.VMEM((2, K), jnp.int32),
        pltpu.VMEM((2, K), jnp.int32),
        pltpu.VMEM((K, HID), jnp.float32),
        pltpu.VMEM((K, HID), jnp.float32),
        pltpu.VMEM((K, HID + 16), jnp.float32),
        pltpu.VMEM((K, 16), jnp.float32),
        pltpu.VMEM((K, 16), jnp.float32),
        pltpu.VMEM((K, 16), jnp.float32),
        pltpu.VMEM((K, 16), jnp.float32),
        pltpu.VMEM((16,), jnp.float32),
        pltpu.VMEM_SHARED((NSP, HID + 16), jnp.float32),
        pltpu.SemaphoreType.DMA,
        pltpu.SemaphoreType.DMA,
    ],
)
def _edge_kernel(hh, a16, cvec, sdm, out,
                 idxp, idxsw, idxdw, idxlw, hbuf0, hbuf1, mbuf,
                 abuf0, abuf1, bbuf0, bbuf1, cbuf,
                 outsp, semg0, semg1):
    cid = lax.axis_index("c")
    sid = lax.axis_index("s")

    zero16 = jnp.zeros((16,), jnp.float32)

    def z_body(k, carry):
        for h in range(NH + 1):
            mbuf[k, pl.ds(h * HD, HD)] = zero16
        return carry

    lax.fori_loop(0, K, z_body, 0)

    pltpu.sync_copy(mbuf.at[pl.ds(0, 128)], outsp.at[pl.ds(sid * RPT, 128)])
    pltpu.sync_copy(mbuf.at[pl.ds(0, 128)],
                    outsp.at[pl.ds(sid * RPT + 128, 128)])
    pltpu.sync_copy(mbuf.at[pl.ds(0, 57)],
                    outsp.at[pl.ds(sid * RPT + 256, 57)])

    pltpu.sync_copy(cvec, cbuf)
    pltpu.sync_copy(sdm.at[cid, pl.ds(sid * CPW, CPW)], idxp)
    plsc.subcore_barrier()

    lo = cid * NH2

    cv = cbuf[:]
    rot8 = jnp.bitwise_xor(lax.iota(jnp.int32, 16), 8)[:, None]
    _gd = lax.GatherDimensionNumbers(
        offset_dims=(), collapsed_slice_dims=(0,), start_index_map=(0,))

    def _perm(v, idx):
        return lax.gather(v, idx, _gd, (1,),
                          mode=lax.GatherScatterMode.PROMISE_IN_BOUNDS)

    def unpack(j, pp):
        for v in range(8):
            p = idxp[j, pl.ds(v * 16, 16)]
            idxsw[pp, pl.ds(v * 16, 16)] = lax.shift_right_logical(p, 14)
            d = jnp.bitwise_and(p, 16383)
            idxdw[pp, pl.ds(v * 16, 16)] = d
            idxlw[pp, pl.ds(v * 16, 16)] = jnp.clip(d - lo, 0, NH2 - 1)

    def issue3(pp, hb, ab, bb, sem):
        pltpu.async_copy(hh.at[idxsw.at[pp]], hb, sem)
        pltpu.async_copy(a16.at[idxsw.at[pp]], ab, sem)
        pltpu.async_copy(a16.at[idxdw.at[pp]], bb, sem)

    def wait3(pp, hb, ab, bb, sem):
        pltpu.make_async_copy(hh.at[idxsw.at[pp]], hb, sem).wait()
        pltpu.make_async_copy(a16.at[idxsw.at[pp]], ab, sem).wait()
        pltpu.make_async_copy(a16.at[idxdw.at[pp]], bb, sem).wait()

    def compute_scatter(pp, hb, ab, bb):
        def m_body(k, c2):
            t = ab[k, :] + _perm(bb[k, :], rot8)
            t = jnp.where(t > 0.0, t, 0.2 * t)
            wv = jnp.exp(t - cv)
            mbuf[k, pl.ds(HID, 16)] = wv
            for h in range(NH):
                mbuf[k, pl.ds(h * HD, HD)] = hb[k, pl.ds(h * HD, HD)] * wv[h]
            return c2

        lax.fori_loop(0, K, m_body, 0)
        pltpu.sync_copy(mbuf, outsp.at[idxlw.at[pp]], add=True)

    unpack(0, 0)
    issue3(0, hbuf0, abuf0, bbuf0, semg0)

    def pipe_body(t, carry):
        j0 = 2 * t
        unpack(j0 + 1, 1)
        issue3(1, hbuf1, abuf1, bbuf1, semg1)
        wait3(0, hbuf0, abuf0, bbuf0, semg0)
        compute_scatter(0, hbuf0, abuf0, bbuf0)
        unpack(jnp.minimum(j0 + 2, CPW - 1), 0)
        issue3(0, hbuf0, abuf0, bbuf0, semg0)
        wait3(1, hbuf1, abuf1, bbuf1, semg1)
        compute_scatter(1, hbuf1, abuf1, bbuf1)
        return carry

    lax.fori_loop(0, CPW // 2, pipe_body, 0)
    wait3(0, hbuf0, abuf0, bbuf0, semg0)
    plsc.subcore_barrier()

    @pl.when(sid < 15)
    def _():
        pltpu.sync_copy(outsp.at[pl.ds(sid * WB, WB)],
                        out.at[cid, pl.ds(sid * WB, WB)])

    @pl.when(sid == 15)
    def _():
        pltpu.sync_copy(outsp.at[pl.ds(15 * WB, NH2 - 15 * WB)],
                        out.at[cid, pl.ds(15 * WB, NH2 - 15 * WB)])


def _pre_common(h, W_ref, AA_ref, hh_ref, as_ref, cv_ref):
    hh = jnp.dot(h, W_ref[:], preferred_element_type=jnp.float32)
    hh_ref[:] = hh
    sc = jnp.dot(hh, AA_ref[:], preferred_element_type=jnp.float32)
    row = lax.broadcasted_iota(jnp.int32, (NP, NH), 0)
    a_s = jnp.where(row < N, sc[:, 0:NH], _NEG)
    a_d = jnp.where(row < N, sc[:, NH:2 * NH], _NEG)
    as_ref[:] = jnp.concatenate([a_s, a_d], axis=1)
    cs = (jnp.max(a_s, axis=0, keepdims=True)
          + jnp.max(a_d, axis=0, keepdims=True))
    cl = jnp.where(cs > 0.0, cs, 0.2 * cs)
    cv_ref[:] = jnp.concatenate([cl, cl], axis=1)


def _embed_pre_body(xp_ref, embp_ref, W_ref, AA_ref,
                    h_ref, hh_ref, as_ref, cv_ref):
    iot = lax.broadcasted_iota(jnp.int32, (NP, 128), 1)
    oh = (xp_ref[:] == iot).astype(jnp.float32)
    h = jnp.dot(oh, embp_ref[:], preferred_element_type=jnp.float32)
    h_ref[:] = h
    _pre_common(h, W_ref, AA_ref, hh_ref, as_ref, cv_ref)


def _norm_block(out2_ref, E16_ref, bias_ref, lg_ref, lb_ref):
    full = jnp.concatenate([out2_ref[0, pl.ds(0, NH2), :],
                            out2_ref[1, pl.ds(0, NH2), :]], axis=0)
    s = full[:, 0:HID]
    d16 = full[:, HID:HID + 16]
    dexp = jnp.dot(d16, E16_ref[:], preferred_element_type=jnp.float32)
    g = s / (dexp + 1e-16) + bias_ref[:]
    mu = jnp.mean(g, axis=-1, keepdims=True)
    var = jnp.mean((g - mu) ** 2, axis=-1, keepdims=True)
    g = (g - mu) / jnp.sqrt(var + 1e-5) * lg_ref[:] + lb_ref[:]
    return jnp.maximum(g, 0.0)


def _make_post_pre_body(has_res):
    def body(out2_ref, E16_ref, bias_ref, lg_ref, lb_ref, res_ref,
             W_ref, AA_ref, h_ref, hh_ref, as_ref, cv_ref):
        g = _norm_block(out2_ref, E16_ref, bias_ref, lg_ref, lb_ref)
        if has_res:
            g = g + res_ref[pl.ds(0, N), :]
        h = jnp.concatenate(
            [g, jnp.zeros((NP - N, HID), jnp.float32)], axis=0)
        h_ref[:] = h
        _pre_common(h, W_ref, AA_ref, hh_ref, as_ref, cv_ref)
    return body


def _post_mlp_body(out2_ref, E16_ref, bias_ref, lg_ref, lb_ref,
                   res_ref, W1_ref, b1_ref, W2_ref, b2_ref, W3_ref, b3_ref,
                   y_ref):
    g = _norm_block(out2_ref, E16_ref, bias_ref, lg_ref, lb_ref)
    g = g + res_ref[pl.ds(0, N), :]
    h1 = jnp.maximum(
        jnp.dot(g, W1_ref[:], preferred_element_type=jnp.float32)
        + b1_ref[:], 0.0)
    h2 = jnp.maximum(
        jnp.dot(h1, W2_ref[:], preferred_element_type=jnp.float32)
        + b2_ref[:], 0.0)
    y_ref[:] = (jnp.dot(h2, W3_ref[:], preferred_element_type=jnp.float32)
                + b3_ref[:])


_f32 = jnp.float32
_node_shapes = [
    jax.ShapeDtypeStruct((NP, HID), _f32),
    jax.ShapeDtypeStruct((NP, HID), _f32),
    jax.ShapeDtypeStruct((NP, 16), _f32),
    jax.ShapeDtypeStruct((1, 16), _f32),
]

_embed_pre = pl.pallas_call(_embed_pre_body, out_shape=_node_shapes)
_post_pre_first = pl.pallas_call(_make_post_pre_body(False),
                                 out_shape=_node_shapes)
_post_pre = pl.pallas_call(_make_post_pre_body(True), out_shape=_node_shapes)
_post_mlp = pl.pallas_call(
    _post_mlp_body, out_shape=jax.ShapeDtypeStruct((N, 128), _f32))

_blk = np.zeros((HID, NH), np.float32)
for _h in range(NH):
    _blk[_h * HD:(_h + 1) * HD, _h] = 1.0
_BLK = _blk
_e16 = np.zeros((16, HID), np.float32)
for _r in range(NH):
    _e16[_r, _r * HD:(_r + 1) * HD] = 1.0
_E16 = _e16


def kernel(x, edge_index, batch, params):
    del batch
    xp = jnp.concatenate(
        [x.astype(jnp.int32), jnp.zeros((NP - N,), jnp.int32)])[:, None]
    loop = jnp.arange(N, dtype=jnp.int32)
    src = jnp.concatenate([edge_index[0].astype(jnp.int32), loop])
    dst = jnp.concatenate([edge_index[1].astype(jnp.int32), loop])
    order = jnp.argsort(dst)
    packed = ((src << 14) | dst)[order]
    c0 = jnp.sum((dst < NH2).astype(jnp.int32))
    j = jnp.arange(packed.shape[0], dtype=jnp.int32)
    slot = jnp.where(j < c0, j, EPH + j - c0)
    base = jnp.where(jnp.arange(2 * EPH) < EPH,
                     (N << 14), (N << 14) | NH2).astype(jnp.int32)
    sdm = base.at[slot].set(packed, unique_indices=True)
    sdm = sdm.reshape(2, NCHUNK, K)
    embp = jnp.zeros((128, 128), _f32).at[:VOCAB].set(params["emb"])

    def expand_att(p):
        As = p["att_src"].reshape(HID)[:, None] * _BLK
        Ad = p["att_dst"].reshape(HID)[:, None] * _BLK
        return jnp.concatenate(
            [As, Ad, jnp.zeros((HID, HID - 2 * NH), _f32)], axis=1)

    layers = params["layers"]
    AA0 = expand_att(layers[0])

    h, hh, a16, cvec = _embed_pre(xp, embp, layers[0]["W"], AA0)

    for i in range(6):
        p = layers[i]
        (out2,) = _edge_kernel(hh, a16, cvec.reshape(16), sdm)
        bias = p["bias"][None, :]
        lg = p["ln_g"][None, :]
        lb = p["ln_b"][None, :]
        if i < 5:
            pn = layers[i + 1]
            AAn = expand_att(pn)
            fn = _post_pre_first if i == 0 else _post_pre
            h, hh, a16, cvec = fn(
                out2, _E16, bias, lg, lb, h, pn["W"], AAn)
        else:
            W3p = jnp.zeros((HID, 128), _f32).at[:, :3].set(params["W3"])
            b3p = jnp.zeros((1, 128), _f32).at[0, :3].set(params["b3"])
            y = _post_mlp(out2, _E16, bias, lg, lb, h,
                          params["W1"], params["b1"][None, :],
                          params["W2"], params["b2"][None, :], W3p, b3p)
    return y[:, :3]


# speedup vs baseline: 1.0430x; 1.0430x over previous
import functools

import jax
import jax.numpy as jnp
import numpy as np
from jax import lax
from jax.experimental import pallas as pl
from jax.experimental.pallas import tpu as pltpu
from jax.experimental.pallas import tpu_sc as plsc

N = 10000
NP = 10016
VOCAB = 100
HID = 128
NH = 8
HD = 16
E_REAL = 330000
K = 128
NCHUNK = 1408
EPH = NCHUNK * K
CPW = NCHUNK // 16
NH2 = N // 2
NSP = NH2 + 8
RPT = 313
WB = 312

_NEG = -1e30


_mesh = plsc.VectorSubcoreMesh(core_axis_name="c", subcore_axis_name="s")


@functools.partial(
    pl.kernel,
    mesh=_mesh,
    compiler_params=pltpu.CompilerParams(use_tc_tiling_on_sc=False),
    out_type=[
        jax.ShapeDtypeStruct((2, NSP, HID + 16), jnp.float32),
    ],
    scratch_types=[
        pltpu.VMEM((CPW, K), jnp.int32),
        pltpu.VMEM((2, K), jnp.int32),
        pltpu.VMEM((2, K), jnp.int32),
        pltpu.VMEM((2, K), jnp.int32),
        pltpu.VMEM((K, HID), jnp.float32),
        pltpu.VMEM((K, HID), jnp.float32),
        pltpu.VMEM((K, HID + 16), jnp.float32),
        pltpu.VMEM((K, 16), jnp.float32),
        pltpu.VMEM((K, 16), jnp.float32),
        pltpu.VMEM((K, 16), jnp.float32),
        pltpu.VMEM((K, 16), jnp.float32),
        pltpu.VMEM((16,), jnp.float32),
        pltpu.VMEM_SHARED((NSP, HID + 16), jnp.float32),
        pltpu.SemaphoreType.DMA,
        pltpu.SemaphoreType.DMA,
    ],
)
def _edge_kernel(hh, a16, cvec, sdm, out,
                 idxp, idxsw, idxdw, idxlw, hbuf0, hbuf1, mbuf,
                 abuf0, abuf1, bbuf0, bbuf1, cbuf,
                 outsp, semg0, semg1):
    cid = lax.axis_index("c")
    sid = lax.axis_index("s")

    zero16 = jnp.zeros((16,), jnp.float32)

    def z_body(k, carry):
        for h in range(NH + 1):
            mbuf[k, pl.ds(h * HD, HD)] = zero16
        return carry

    lax.fori_loop(0, K, z_body, 0)

    pltpu.sync_copy(mbuf.at[pl.ds(0, 128)], outsp.at[pl.ds(sid * RPT, 128)])
    pltpu.sync_copy(mbuf.at[pl.ds(0, 128)],
                    outsp.at[pl.ds(sid * RPT + 128, 128)])
    pltpu.sync_copy(mbuf.at[pl.ds(0, 57)],
                    outsp.at[pl.ds(sid * RPT + 256, 57)])

    pltpu.sync_copy(cvec, cbuf)
    pltpu.sync_copy(sdm.at[cid, pl.ds(sid * CPW, CPW)], idxp)
    plsc.subcore_barrier()

    lo = cid * NH2

    cv = cbuf[:]
    _HIDX = [jnp.full((16, 1), h, jnp.int32) for h in range(NH)]
    rot8 = jnp.bitwise_xor(lax.iota(jnp.int32, 16), 8)[:, None]
    _gd = lax.GatherDimensionNumbers(
        offset_dims=(), collapsed_slice_dims=(0,), start_index_map=(0,))

    def _perm(v, idx):
        return lax.gather(v, idx, _gd, (1,),
                          mode=lax.GatherScatterMode.PROMISE_IN_BOUNDS)

    def unpack(j, pp):
        for v in range(8):
            p = idxp[j, pl.ds(v * 16, 16)]
            idxsw[pp, pl.ds(v * 16, 16)] = lax.shift_right_logical(p, 14)
            d = jnp.bitwise_and(p, 16383)
            idxdw[pp, pl.ds(v * 16, 16)] = d
            idxlw[pp, pl.ds(v * 16, 16)] = jnp.clip(d - lo, 0, NH2 - 1)

    def issue3(pp, hb, ab, bb, sem):
        pltpu.async_copy(hh.at[idxsw.at[pp]], hb, sem)
        pltpu.async_copy(a16.at[idxsw.at[pp]], ab, sem)
        pltpu.async_copy(a16.at[idxdw.at[pp]], bb, sem)

    def wait3(pp, hb, ab, bb, sem):
        pltpu.make_async_copy(hh.at[idxsw.at[pp]], hb, sem).wait()
        pltpu.make_async_copy(a16.at[idxsw.at[pp]], ab, sem).wait()
        pltpu.make_async_copy(a16.at[idxdw.at[pp]], bb, sem).wait()

    def compute_scatter(pp, hb, ab, bb):
        def m_body(k, c2):
            t = ab[k, :] + _perm(bb[k, :], rot8)
            t = jnp.where(t > 0.0, t, 0.2 * t)
            wv = jnp.exp(t - cv)
            mbuf[k, pl.ds(HID, 16)] = wv
            for h in range(NH):
                wb = _perm(wv, _HIDX[h])
                mbuf[k, pl.ds(h * HD, HD)] = hb[k, pl.ds(h * HD, HD)] * wb
            return c2

        lax.fori_loop(0, K, m_body, 0)
        pltpu.sync_copy(mbuf, outsp.at[idxlw.at[pp]], add=True)

    unpack(0, 0)
    issue3(0, hbuf0, abuf0, bbuf0, semg0)

    def pipe_body(t, carry):
        j0 = 2 * t
        unpack(j0 + 1, 1)
        issue3(1, hbuf1, abuf1, bbuf1, semg1)
        wait3(0, hbuf0, abuf0, bbuf0, semg0)
        compute_scatter(0, hbuf0, abuf0, bbuf0)
        unpack(jnp.minimum(j0 + 2, CPW - 1), 0)
        issue3(0, hbuf0, abuf0, bbuf0, semg0)
        wait3(1, hbuf1, abuf1, bbuf1, semg1)
        compute_scatter(1, hbuf1, abuf1, bbuf1)
        return carry

    lax.fori_loop(0, CPW // 2, pipe_body, 0)
    wait3(0, hbuf0, abuf0, bbuf0, semg0)
    plsc.subcore_barrier()

    @pl.when(sid < 15)
    def _():
        pltpu.sync_copy(outsp.at[pl.ds(sid * WB, WB)],
                        out.at[cid, pl.ds(sid * WB, WB)])

    @pl.when(sid == 15)
    def _():
        pltpu.sync_copy(outsp.at[pl.ds(15 * WB, NH2 - 15 * WB)],
                        out.at[cid, pl.ds(15 * WB, NH2 - 15 * WB)])


def _pre_common(h, W_ref, AA_ref, hh_ref, as_ref, cv_ref):
    hh = jnp.dot(h, W_ref[:], preferred_element_type=jnp.float32)
    hh_ref[:] = hh
    sc = jnp.dot(hh, AA_ref[:], preferred_element_type=jnp.float32)
    row = lax.broadcasted_iota(jnp.int32, (NP, NH), 0)
    a_s = jnp.where(row < N, sc[:, 0:NH], _NEG)
    a_d = jnp.where(row < N, sc[:, NH:2 * NH], _NEG)
    as_ref[:] = jnp.concatenate([a_s, a_d], axis=1)
    cs = (jnp.max(a_s, axis=0, keepdims=True)
          + jnp.max(a_d, axis=0, keepdims=True))
    cl = jnp.where(cs > 0.0, cs, 0.2 * cs)
    cv_ref[:] = jnp.concatenate([cl, cl], axis=1)


def _embed_pre_body(xp_ref, embp_ref, W_ref, AA_ref,
                    h_ref, hh_ref, as_ref, cv_ref):
    iot = lax.broadcasted_iota(jnp.int32, (NP, 128), 1)
    oh = (xp_ref[:] == iot).astype(jnp.float32)
    h = jnp.dot(oh, embp_ref[:], preferred_element_type=jnp.float32)
    h_ref[:] = h
    _pre_common(h, W_ref, AA_ref, hh_ref, as_ref, cv_ref)


def _norm_block(out2_ref, E16_ref, bias_ref, lg_ref, lb_ref):
    full = jnp.concatenate([out2_ref[0, pl.ds(0, NH2), :],
                            out2_ref[1, pl.ds(0, NH2), :]], axis=0)
    s = full[:, 0:HID]
    d16 = full[:, HID:HID + 16]
    dexp = jnp.dot(d16, E16_ref[:], preferred_element_type=jnp.float32)
    g = s / (dexp + 1e-16) + bias_ref[:]
    mu = jnp.mean(g, axis=-1, keepdims=True)
    var = jnp.mean((g - mu) ** 2, axis=-1, keepdims=True)
    g = (g - mu) / jnp.sqrt(var + 1e-5) * lg_ref[:] + lb_ref[:]
    return jnp.maximum(g, 0.0)


def _make_post_pre_body(has_res):
    def body(out2_ref, E16_ref, bias_ref, lg_ref, lb_ref, res_ref,
             W_ref, AA_ref, h_ref, hh_ref, as_ref, cv_ref):
        g = _norm_block(out2_ref, E16_ref, bias_ref, lg_ref, lb_ref)
        if has_res:
            g = g + res_ref[pl.ds(0, N), :]
        h = jnp.concatenate(
            [g, jnp.zeros((NP - N, HID), jnp.float32)], axis=0)
        h_ref[:] = h
        _pre_common(h, W_ref, AA_ref, hh_ref, as_ref, cv_ref)
    return body


def _post_mlp_body(out2_ref, E16_ref, bias_ref, lg_ref, lb_ref,
                   res_ref, W1_ref, b1_ref, W2_ref, b2_ref, W3_ref, b3_ref,
                   y_ref):
    g = _norm_block(out2_ref, E16_ref, bias_ref, lg_ref, lb_ref)
    g = g + res_ref[pl.ds(0, N), :]
    h1 = jnp.maximum(
        jnp.dot(g, W1_ref[:], preferred_element_type=jnp.float32)
        + b1_ref[:], 0.0)
    h2 = jnp.maximum(
        jnp.dot(h1, W2_ref[:], preferred_element_type=jnp.float32)
        + b2_ref[:], 0.0)
    y_ref[:] = (jnp.dot(h2, W3_ref[:], preferred_element_type=jnp.float32)
                + b3_ref[:])


_f32 = jnp.float32
_node_shapes = [
    jax.ShapeDtypeStruct((NP, HID), _f32),
    jax.ShapeDtypeStruct((NP, HID), _f32),
    jax.ShapeDtypeStruct((NP, 16), _f32),
    jax.ShapeDtypeStruct((1, 16), _f32),
]

_embed_pre = pl.pallas_call(_embed_pre_body, out_shape=_node_shapes)
_post_pre_first = pl.pallas_call(_make_post_pre_body(False),
                                 out_shape=_node_shapes)
_post_pre = pl.pallas_call(_make_post_pre_body(True), out_shape=_node_shapes)
_post_mlp = pl.pallas_call(
    _post_mlp_body, out_shape=jax.ShapeDtypeStruct((N, 128), _f32))

_blk = np.zeros((HID, NH), np.float32)
for _h in range(NH):
    _blk[_h * HD:(_h + 1) * HD, _h] = 1.0
_BLK = _blk
_e16 = np.zeros((16, HID), np.float32)
for _r in range(NH):
    _e16[_r, _r * HD:(_r + 1) * HD] = 1.0
_E16 = _e16


def kernel(x, edge_index, batch, params):
    del batch
    xp = jnp.concatenate(
        [x.astype(jnp.int32), jnp.zeros((NP - N,), jnp.int32)])[:, None]
    loop = jnp.arange(N, dtype=jnp.int32)
    src = jnp.concatenate([edge_index[0].astype(jnp.int32), loop])
    dst = jnp.concatenate([edge_index[1].astype(jnp.int32), loop])
    h0 = (dst < NH2).astype(jnp.int32)
    c0 = jnp.cumsum(h0)
    c1 = jnp.cumsum(1 - h0)
    slot = jnp.where(h0 == 1, c0 - 1, EPH + c1 - 1)
    packed = (src << 14) | dst
    base = jnp.where(jnp.arange(2 * EPH) < EPH,
                     (N << 14), (N << 14) | NH2).astype(jnp.int32)
    sdm = base.at[slot].set(packed, unique_indices=True)
    sdm = sdm.reshape(2, NCHUNK, K)
    embp = jnp.zeros((128, 128), _f32).at[:VOCAB].set(params["emb"])

    def expand_att(p):
        As = p["att_src"].reshape(HID)[:, None] * _BLK
        Ad = p["att_dst"].reshape(HID)[:, None] * _BLK
        return jnp.concatenate(
            [As, Ad, jnp.zeros((HID, HID - 2 * NH), _f32)], axis=1)

    layers = params["layers"]
    AA0 = expand_att(layers[0])

    h, hh, a16, cvec = _embed_pre(xp, embp, layers[0]["W"], AA0)

    for i in range(6):
        p = layers[i]
        (out2,) = _edge_kernel(hh, a16, cvec.reshape(16), sdm)
        bias = p["bias"][None, :]
        lg = p["ln_g"][None, :]
        lb = p["ln_b"][None, :]
        if i < 5:
            pn = layers[i + 1]
            AAn = expand_att(pn)
            fn = _post_pre_first if i == 0 else _post_pre
            h, hh, a16, cvec = fn(
                out2, _E16, bias, lg, lb, h, pn["W"], AAn)
        else:
            W3p = jnp.zeros((HID, 128), _f32).at[:, :3].set(params["W3"])
            b3p = jnp.zeros((1, 128), _f32).at[0, :3].set(params["b3"])
            y = _post_mlp(out2, _E16, bias, lg, lb, h,
                          params["W1"], params["b1"][None, :],
                          params["W2"], params["b2"][None, :], W3p, b3p)
    return y[:, :3]
